# baseline (speedup 1.0000x reference)
import functools
import math

import jax
import jax.numpy as jnp
from jax import lax
from jax.experimental import pallas as pl
from jax.experimental.pallas import tpu as pltpu
from jax.experimental.pallas import tpu_sc as plsc

D_MODEL = 64
MAXLEN = 200
BATCH = 4096
N_ROWS = BATCH * MAXLEN
NC, NS = 2, 16
NW = NC * NS
ROWS_PER_W = N_ROWS // NW
CHUNK = 128
NCHUNK = ROWS_PER_W // CHUNK
SCALE = math.sqrt(D_MODEL)
NSLICE = D_MODEL // 16

_mesh = plsc.VectorSubcoreMesh(core_axis_name="c", subcore_axis_name="s")


@functools.partial(
    pl.kernel,
    out_type=jax.ShapeDtypeStruct((N_ROWS, D_MODEL), jnp.float32),
    mesh=_mesh,
    scratch_types=[
        pltpu.VMEM((MAXLEN, D_MODEL), jnp.float32),
        pltpu.VMEM((CHUNK,), jnp.int32),
        pltpu.VMEM((CHUNK, D_MODEL), jnp.float32),
        pltpu.SemaphoreType.DMA,
    ],
    compiler_params=pltpu.CompilerParams(use_tc_tiling_on_sc=False),
)
def _emb(table_hbm, idx_hbm, pos_hbm, out_hbm, pos_v, idx_v, rows_v, sem):
    wid = lax.axis_index("s") * NC + lax.axis_index("c")
    base = wid * ROWS_PER_W

    pltpu.sync_copy(pos_hbm, pos_v)

    def scale_pos(t, carry):
        for j in range(NSLICE):
            sl = pl.ds(j * 16, 16)
            pos_v[t, sl] = pos_v[t, sl] * SCALE
        return carry

    lax.fori_loop(0, MAXLEN, scale_pos, 0)

    def chunk_body(c, carry):
        r0 = base + c * CHUNK
        pltpu.sync_copy(idx_hbm.at[pl.ds(r0, CHUNK)], idx_v)
        pltpu.async_copy(table_hbm.at[idx_v], rows_v, sem).wait()
        phase0 = lax.rem(r0, MAXLEN)

        def row_body(i, rcarry):
            t = lax.rem(phase0 + i, MAXLEN)
            for j in range(NSLICE):
                sl = pl.ds(j * 16, 16)
                rows_v[i, sl] = rows_v[i, sl] * SCALE + pos_v[t, sl]
            return rcarry

        lax.fori_loop(0, CHUNK, row_body, 0)
        pltpu.sync_copy(rows_v, out_hbm.at[pl.ds(r0, CHUNK)])
        return carry

    lax.fori_loop(0, NCHUNK, chunk_body, 0)


def kernel(x, token_table, pos_table):
    xf = x.reshape(N_ROWS).astype(jnp.int32)
    out = _emb(token_table, xf, pos_table)
    return out.reshape(BATCH, MAXLEN, D_MODEL)


# speedup vs baseline: 1.5426x; 1.5426x over previous
import functools
import math

import jax
import jax.numpy as jnp
from jax import lax
from jax.experimental import pallas as pl
from jax.experimental.pallas import tpu as pltpu
from jax.experimental.pallas import tpu_sc as plsc

D_MODEL = 64
MAXLEN = 200
BATCH = 4096
N_ROWS = BATCH * MAXLEN
NC, NS = 2, 16
NW = NC * NS
ROWS_PER_W = N_ROWS // NW
NSEQ = ROWS_PER_W // MAXLEN
SCALE = math.sqrt(D_MODEL)
NSLICE = D_MODEL // 16
NBUF = 4
LOOK = 2
G1, G2 = 104, 96

_mesh = plsc.VectorSubcoreMesh(core_axis_name="c", subcore_axis_name="s")


@functools.partial(
    pl.kernel,
    out_type=jax.ShapeDtypeStruct((N_ROWS, D_MODEL), jnp.float32),
    mesh=_mesh,
    scratch_types=(
        [pltpu.VMEM((MAXLEN, D_MODEL), jnp.float32)]
        + [pltpu.VMEM((ROWS_PER_W,), jnp.int32)]
        + [pltpu.VMEM((MAXLEN, D_MODEL), jnp.float32)] * NBUF
        + [pltpu.SemaphoreType.DMA] * NBUF
        + [pltpu.SemaphoreType.DMA] * NBUF
    ),
    compiler_params=pltpu.CompilerParams(use_tc_tiling_on_sc=False),
)
def _emb(table_hbm, idx_hbm, pos_hbm, out_hbm, pos_v, idx_v, *bufs_and_sems):
    rows = bufs_and_sems[:NBUF]
    sem_g = bufs_and_sems[NBUF:2 * NBUF]
    sem_o = bufs_and_sems[2 * NBUF:3 * NBUF]

    wid = lax.axis_index("s") * NC + lax.axis_index("c")
    base = wid * ROWS_PER_W

    pltpu.sync_copy(idx_hbm.at[pl.ds(base, ROWS_PER_W)], idx_v)
    pltpu.sync_copy(pos_hbm, pos_v)

    def scale_pos(t, carry):
        for j in range(NSLICE):
            sl = pl.ds(j * 16, 16)
            pos_v[t, sl] = pos_v[t, sl] * SCALE
        return carry

    lax.fori_loop(0, MAXLEN, scale_pos, 0)

    def start_gather(c, b):
        off = c * MAXLEN
        pltpu.async_copy(table_hbm.at[idx_v.at[pl.ds(off, G1)]],
                         rows[b].at[pl.ds(0, G1)], sem_g[b])
        pltpu.async_copy(table_hbm.at[idx_v.at[pl.ds(off + G1, G2)]],
                         rows[b].at[pl.ds(G1, G2)], sem_g[b])

    def wait_gather(b):
        pltpu.make_async_copy(table_hbm.at[idx_v.at[pl.ds(0, G1)]],
                              rows[b].at[pl.ds(0, G1)], sem_g[b]).wait()
        pltpu.make_async_copy(table_hbm.at[idx_v.at[pl.ds(0, G2)]],
                              rows[b].at[pl.ds(G1, G2)], sem_g[b]).wait()

    def start_out(c, b):
        pltpu.async_copy(rows[b], out_hbm.at[pl.ds(base + c * MAXLEN, MAXLEN)],
                         sem_o[b])

    def wait_out(b):
        pltpu.make_async_copy(rows[b], out_hbm.at[pl.ds(0, MAXLEN)],
                              sem_o[b]).wait()

    def compute(b):
        buf = rows[b]

        def row_body(t, carry):
            for j in range(NSLICE):
                sl = pl.ds(j * 16, 16)
                buf[t, sl] = buf[t, sl] * SCALE + pos_v[t, sl]
            return carry

        lax.fori_loop(0, MAXLEN, row_body, 0)

    start_gather(0, 0)
    start_gather(1, 1)

    for c in range(2):
        start_gather(c + LOOK, (c + LOOK) % NBUF)
        wait_gather(c % NBUF)
        compute(c % NBUF)
        start_out(c, c % NBUF)
    for c in range(2, 4):
        wait_out((c + LOOK) % NBUF)
        start_gather(c + LOOK, (c + LOOK) % NBUF)
        wait_gather(c % NBUF)
        compute(c % NBUF)
        start_out(c, c % NBUF)

    def steady(i, carry):
        c0 = i * NBUF
        for k in range(NBUF):
            c = c0 + k
            wait_out((k + LOOK) % NBUF)
            start_gather(c + LOOK, (k + LOOK) % NBUF)
            wait_gather(k)
            compute(k)
            start_out(c, k)
        return carry

    lax.fori_loop(1, NSEQ // NBUF - 1, steady, 0)

    for c in range(NSEQ - 4, NSEQ - 2):
        b = c % NBUF
        wait_out((b + LOOK) % NBUF)
        start_gather(c + LOOK, (b + LOOK) % NBUF)
        wait_gather(b)
        compute(b)
        start_out(c, b)
    for c in range(NSEQ - 2, NSEQ):
        b = c % NBUF
        wait_gather(b)
        compute(b)
        start_out(c, b)
    for b in range(NBUF):
        wait_out(b)


def kernel(x, token_table, pos_table):
    xf = x.reshape(N_ROWS).astype(jnp.int32)
    out = _emb(token_table, xf, pos_table)
    return out.reshape(BATCH, MAXLEN, D_MODEL)
